# Initial kernel scaffold; baseline (speedup 1.0000x reference)
import jax
import jax.numpy as jnp
from jax.experimental import pallas as pl


def kernel(skip, x, w1, w3, s1, b1, s3, b3):
    raise NotImplementedError("write your pallas kernel here")


# speedup vs baseline: 1.9490x; 1.9490x over previous
import functools

import numpy as np

import jax
import jax.numpy as jnp
from jax.experimental import pallas as pl
from jax.experimental.pallas import tpu as pltpu


def _bilin_taps(out_size, in_size):
    scale = in_size / out_size
    i0s, i1s, l0s, l1s = [], [], [], []
    for o in range(out_size):
        src = max((o + 0.5) * scale - 0.5, 0.0)
        i0 = min(int(np.floor(src)), in_size - 1)
        i1 = min(i0 + 1, in_size - 1)
        l1 = src - i0
        i0s.append(i0)
        i1s.append(i1)
        l0s.append(1.0 - l1)
        l1s.append(l1)
    return i0s, i1s, l0s, l1s


def _bilin_matrix_T(out_size, in_size):
    i0s, i1s, l0s, l1s = _bilin_taps(out_size, in_size)
    m = np.zeros((in_size, out_size), np.float32)
    for o in range(out_size):
        m[i0s[o], o] += l0s[o]
        m[i1s[o], o] += l1s[o]
    return m


def _wresize_kernel(x_ref, w_ref, o_ref):
    xb = x_ref[...].astype(jnp.bfloat16)
    o_ref[...] = jnp.dot(xb, w_ref[...],
                         preferred_element_type=jnp.float32).astype(o_ref.dtype)


def _wresize(x2d, wwT_b, tr=2048):
    R, Wx = x2d.shape
    _, Ws = wwT_b.shape
    tr = min(tr, R)
    Rp = ((R + tr - 1) // tr) * tr
    if Rp != R:
        x2d = jnp.pad(x2d, ((0, Rp - R), (0, 0)))
    out = pl.pallas_call(
        _wresize_kernel,
        out_shape=jax.ShapeDtypeStruct((Rp, Ws), jnp.bfloat16),
        grid_spec=pltpu.PrefetchScalarGridSpec(
            num_scalar_prefetch=0,
            grid=(Rp // tr,),
            in_specs=[
                pl.BlockSpec((tr, Wx), lambda i: (i, 0)),
                pl.BlockSpec((Wx, Ws), lambda i: (0, 0)),
            ],
            out_specs=pl.BlockSpec((tr, Ws), lambda i: (i, 0)),
        ),
        compiler_params=pltpu.CompilerParams(dimension_semantics=("parallel",)),
    )(x2d, wwT_b)
    return out[:R]


def _fused_kernel(skip_ref, t_ref, w1s_ref, b1_ref, w3b_ref, b3_ref, mask_ref,
                  o_ref, ypad_ref, *, hs_taps, ws, margin):
    c, s = o_ref.shape
    h0s, h1s, l0s, l1s = hs_taps

    sb = jnp.dot(w1s_ref[...], skip_ref[...],
                 preferred_element_type=jnp.float32)
    sb = jnp.maximum(sb + b1_ref[...], 0.0)

    ypad_ref[:, :margin] = jnp.zeros((c, margin), ypad_ref.dtype)
    ypad_ref[:, s + margin:] = jnp.zeros((c, margin), ypad_ref.dtype)

    for hs in range(len(h0s)):
        a = t_ref[:, h0s[hs] * ws:(h0s[hs] + 1) * ws].astype(jnp.float32)
        b = t_ref[:, h1s[hs] * ws:(h1s[hs] + 1) * ws].astype(jnp.float32)
        y = l0s[hs] * a + l1s[hs] * b + sb[:, hs * ws:(hs + 1) * ws]
        ypad_ref[:, margin + hs * ws:margin + (hs + 1) * ws] = (
            y.astype(ypad_ref.dtype))

    acc = jnp.zeros((c, s), jnp.float32)
    for k in range(9):
        dy = k // 3 - 1
        dx = k % 3 - 1
        d = dy * ws + dx
        ys = ypad_ref[:, pl.ds(margin + d, s)]
        if dx == -1:
            ys = ys * mask_ref[0]
        elif dx == 1:
            ys = ys * mask_ref[1]
        acc = acc + jnp.dot(w3b_ref[k], ys,
                            preferred_element_type=jnp.float32)

    o_ref[...] = jnp.maximum(acc + b3_ref[...], 0.0)


def kernel(skip, x, w1, w3, s1, b1, s3, b3):
    N, Cskip, Hs, Ws = skip.shape
    _, C, Hx, Wx = x.shape
    S = Hs * Ws
    margin = 128

    wwT_b = jnp.asarray(_bilin_matrix_T(Ws, Wx), dtype=jnp.bfloat16)
    hs_taps = _bilin_taps(Hs, Hx)

    w1s = w1 * s1[:, None]
    w3b = (jnp.transpose(w3 * s3[:, None, None, None], (2, 3, 0, 1))
           .reshape(9, C, C).astype(jnp.bfloat16))
    b1c = b1[:, None]
    b3c = b3[:, None]

    ws_idx = np.arange(S) % Ws
    mask = np.stack([(ws_idx != 0), (ws_idx != Ws - 1)]).astype(np.float32)
    mask = jnp.asarray(mask.reshape(2, 1, S), dtype=jnp.bfloat16)

    t = _wresize(x.reshape(N * C * Hx, Wx), wwT_b)
    t3 = t.reshape(N, C, Hx * Ws)

    skip2 = skip.reshape(N, Cskip, S)

    out = pl.pallas_call(
        functools.partial(_fused_kernel, hs_taps=hs_taps, ws=Ws,
                          margin=margin),
        out_shape=jax.ShapeDtypeStruct((N, C, S), jnp.float32),
        grid_spec=pltpu.PrefetchScalarGridSpec(
            num_scalar_prefetch=0,
            grid=(N,),
            in_specs=[
                pl.BlockSpec((None, Cskip, S), lambda n: (n, 0, 0)),
                pl.BlockSpec((None, C, Hx * Ws), lambda n: (n, 0, 0)),
                pl.BlockSpec((C, Cskip), lambda n: (0, 0)),
                pl.BlockSpec((C, 1), lambda n: (0, 0)),
                pl.BlockSpec((9, C, C), lambda n: (0, 0, 0)),
                pl.BlockSpec((C, 1), lambda n: (0, 0)),
                pl.BlockSpec((2, 1, S), lambda n: (0, 0, 0)),
            ],
            out_specs=pl.BlockSpec((None, C, S), lambda n: (n, 0, 0)),
            scratch_shapes=[pltpu.VMEM((C, S + 2 * margin), jnp.bfloat16)],
        ),
        compiler_params=pltpu.CompilerParams(dimension_semantics=("parallel",)),
    )(skip2, t3, w1s, b1c, w3b, b3c, mask)

    return out.reshape(N, C, Hs, Ws)


# speedup vs baseline: 2.6988x; 1.3847x over previous
import functools

import numpy as np

import jax
import jax.numpy as jnp
from jax.experimental import pallas as pl
from jax.experimental.pallas import tpu as pltpu


def _bilin_matrix(out_size, in_size):
    scale = in_size / out_size
    m = np.zeros((out_size, in_size), np.float32)
    for o in range(out_size):
        src = max((o + 0.5) * scale - 0.5, 0.0)
        i0 = min(int(np.floor(src)), in_size - 1)
        i1 = min(i0 + 1, in_size - 1)
        l1 = src - i0
        m[o, i0] += 1.0 - l1
        m[o, i1] += l1
    return m


def _fused_kernel(x_ref, skip_ref, m_ref, w1s_ref, b1_ref, w3c_ref, b3_ref,
                  mask_ref, o_ref, ypad_ref, y9_ref, *, ws, margin):
    c, s = o_ref.shape
    hxwx = m_ref.shape[0]

    xb = x_ref[...].reshape(c, hxwx).astype(jnp.bfloat16)
    up = jnp.dot(xb, m_ref[...], preferred_element_type=jnp.float32)

    sb = jnp.dot(w1s_ref[...], skip_ref[...],
                 preferred_element_type=jnp.float32)
    sb = jnp.maximum(sb + b1_ref[...], 0.0)

    ypad_ref[:, :margin] = jnp.zeros((c, margin), ypad_ref.dtype)
    ypad_ref[:, s + margin:] = jnp.zeros((c, margin), ypad_ref.dtype)
    ypad_ref[:, margin:s + margin] = (up + sb).astype(ypad_ref.dtype)

    for k in range(9):
        dy = k // 3 - 1
        dx = k % 3 - 1
        d = dy * ws + dx
        ys = ypad_ref[:, pl.ds(margin + d, s)]
        if dx == -1:
            ys = ys * mask_ref[0]
        elif dx == 1:
            ys = ys * mask_ref[1]
        y9_ref[k * c:(k + 1) * c, :] = ys

    acc = jnp.dot(w3c_ref[...], y9_ref[...],
                  preferred_element_type=jnp.float32)
    o_ref[...] = jnp.maximum(acc + b3_ref[...], 0.0)


def kernel(skip, x, w1, w3, s1, b1, s3, b3):
    N, Cskip, Hs, Ws = skip.shape
    _, C, Hx, Wx = x.shape
    S = Hs * Ws
    margin = 128

    wh = _bilin_matrix(Hs, Hx)
    ww = _bilin_matrix(Ws, Wx)
    mfull = np.einsum("oi,pj->ijop", wh, ww).reshape(Hx * Wx, S)
    mfull = jnp.asarray(mfull, dtype=jnp.bfloat16)

    w1s = w1 * s1[:, None]
    w3c = (jnp.transpose(w3 * s3[:, None, None, None], (0, 2, 3, 1))
           .reshape(C, 9 * C).astype(jnp.bfloat16))
    b1c = b1[:, None]
    b3c = b3[:, None]

    ws_idx = np.arange(S) % Ws
    mask = np.stack([(ws_idx != 0), (ws_idx != Ws - 1)]).astype(np.float32)
    mask = jnp.asarray(mask.reshape(2, 1, S), dtype=jnp.bfloat16)

    skip2 = skip.reshape(N, Cskip, S)

    out = pl.pallas_call(
        functools.partial(_fused_kernel, ws=Ws, margin=margin),
        out_shape=jax.ShapeDtypeStruct((N, C, S), jnp.float32),
        grid_spec=pltpu.PrefetchScalarGridSpec(
            num_scalar_prefetch=0,
            grid=(N,),
            in_specs=[
                pl.BlockSpec((None, C, Hx, Wx), lambda n: (n, 0, 0, 0)),
                pl.BlockSpec((None, Cskip, S), lambda n: (n, 0, 0)),
                pl.BlockSpec((Hx * Wx, S), lambda n: (0, 0)),
                pl.BlockSpec((C, Cskip), lambda n: (0, 0)),
                pl.BlockSpec((C, 1), lambda n: (0, 0)),
                pl.BlockSpec((C, 9 * C), lambda n: (0, 0)),
                pl.BlockSpec((C, 1), lambda n: (0, 0)),
                pl.BlockSpec((2, 1, S), lambda n: (0, 0, 0)),
            ],
            out_specs=pl.BlockSpec((None, C, S), lambda n: (n, 0, 0)),
            scratch_shapes=[pltpu.VMEM((C, S + 2 * margin), jnp.bfloat16),
                            pltpu.VMEM((9 * C, S), jnp.bfloat16)],
        ),
        compiler_params=pltpu.CompilerParams(dimension_semantics=("parallel",)),
    )(x, skip2, mfull, w1s, b1c, w3c, b3c, mask)

    return out.reshape(N, C, Hs, Ws)
